# baseline (speedup 1.0000x reference)
import functools

import jax
import jax.numpy as jnp
from jax import lax
from jax.experimental import pallas as pl
from jax.experimental.pallas import tpu as pltpu
from jax.experimental.pallas import tpu_sc as plsc

LEN_RULES = 100000
NUM_REL_RULES = 50000
PAD = NUM_REL_RULES
BATCH = 16384
HIST = 200

NC = 2
NS = 16
NW = NC * NS
L = 16

G2L_PAD = 100352
G2L_SLICE = G2L_PAD // NS
TABLE_PAD = 50048
PAD_IDX = 100001
G2L_LAST = 5888
G2L_REST = 32

TBL_SLICE = 3200
TC = 32
NCH = TBL_SLICE // TC
NCH15 = 62

ROWS_PER_W = BATCH // NW
GROUP_ROWS = 16
GROUPS = ROWS_PER_W // GROUP_ROWS
GROUP_WORDS = GROUP_ROWS * HIST

_mesh = plsc.VectorSubcoreMesh(core_axis_name="c", subcore_axis_name="s")


@functools.partial(
    pl.kernel,
    mesh=_mesh,
    compiler_params=pltpu.CompilerParams(needs_layout_passes=False),
    out_type=jax.ShapeDtypeStruct((BATCH,), jnp.float32),
    scratch_types=[
        pltpu.VMEM((G2L_PAD,), jnp.float32),
        pltpu.VMEM((GROUP_ROWS, HIST), jnp.int32),
        pltpu.VMEM((GROUP_ROWS, HIST), jnp.int32),
        pltpu.VMEM((G2L_SLICE,), jnp.int32),
        pltpu.VMEM((ROWS_PER_W,), jnp.float32),
        pltpu.VMEM((L,), jnp.float32),
        pltpu.VMEM((L * L,), jnp.float32),
        pltpu.VMEM((TC, 1), jnp.float32),
        pltpu.VMEM((TC, 1), jnp.float32),
        pltpu.VMEM((G2L_REST,), jnp.int32),
        pltpu.VMEM_SHARED((G2L_PAD,), jnp.float32),
        pltpu.SemaphoreType.DMA,
        pltpu.SemaphoreType.DMA,
        pltpu.SemaphoreType.DMA,
        pltpu.SemaphoreType.DMA,
    ],
)
def _agg(rules_hbm, g2l_hbm, table_hbm, bias_hbm, out_hbm,
         big_v, rbuf0, rbuf1, g2l_v, out_v, bias_v, amat, tcol0, tcol1,
         gtail, shared, sem0, sem1, tsem0, tsem1):
    c = lax.axis_index("c")
    s = lax.axis_index("s")
    wid = s * NC + c
    lane = lax.iota(jnp.int32, L)
    base_row = wid * ROWS_PER_W
    sems = (sem0, sem1)
    rbufs = (rbuf0, rbuf1)

    def rules_dma(g, b):
        start = base_row + g * GROUP_ROWS
        return pltpu.async_copy(rules_hbm.at[pl.ds(start, GROUP_ROWS)],
                                rbufs[b], sems[b])

    handles = [rules_dma(0, 0), rules_dma(1, 1)]

    tbase = s * TBL_SLICE
    tcols = (tcol0, tcol1)
    tsems = (tsem0, tsem1)
    zero16 = jnp.zeros((L,), jnp.int32)

    def tchunk_src(k):
        return table_hbm.at[pl.ds(tbase + k * TC, TC)]

    def tchunk_compact(k, b2):
        pltpu.make_async_copy(tchunk_src(k), tcols[b2], tsems[b2]).wait()
        for h in range(TC // L):
            vals = plsc.load_gather(tcols[b2], [lane + h * L, zero16])
            big_v[pl.ds(tbase + k * TC + h * L, L)] = vals

    def delayout(nch):
        pltpu.async_copy(tchunk_src(0), tcol0, tsem0)
        pltpu.async_copy(tchunk_src(1), tcol1, tsem1)

        def chunk_pair(k2, carry):
            for b2 in (0, 1):
                k = k2 * 2 + b2
                tchunk_compact(k, b2)

                @pl.when(k2 < nch // 2 - 1)
                def _():
                    pltpu.async_copy(tchunk_src(k + 2), tcols[b2],
                                     tsems[b2])
            return carry

        lax.fori_loop(0, nch // 2, chunk_pair, 0)

    @pl.when(s < NS - 1)
    def _():
        delayout(NCH)

    @pl.when(s == NS - 1)
    def _():
        delayout(NCH15)
        pltpu.sync_copy(table_hbm.at[pl.ds(tbase + NCH15 * TC, L)],
                        tcol0.at[pl.ds(0, L)])
        vals = plsc.load_gather(tcol0, [lane, zero16])
        big_v[pl.ds(tbase + NCH15 * TC, L)] = vals

    pltpu.sync_copy(big_v.at[pl.ds(tbase, TBL_SLICE)],
                    shared.at[pl.ds(tbase, TBL_SLICE)])
    plsc.subcore_barrier()
    pltpu.sync_copy(shared.at[pl.ds(0, TABLE_PAD)],
                    big_v.at[pl.ds(0, TABLE_PAD)])
    plsc.subcore_barrier()

    @pl.when(s < NS - 1)
    def _():
        pltpu.sync_copy(g2l_hbm.at[pl.ds(s * G2L_SLICE, G2L_SLICE)], g2l_v)

    @pl.when(s == NS - 1)
    def _():
        base = (NS - 1) * G2L_SLICE
        pltpu.sync_copy(g2l_hbm.at[pl.ds(base, G2L_LAST)],
                        g2l_v.at[pl.ds(0, G2L_LAST)])
        pltpu.sync_copy(g2l_hbm.at[pl.ds(base + G2L_LAST, G2L_REST)], gtail)
        for k in range(G2L_REST // L):
            g2l_v[pl.ds(G2L_LAST + k * L, L)] = gtail[pl.ds(k * L, L)]
        padv = jnp.full((L,), PAD, jnp.int32)
        for k in range((G2L_SLICE - G2L_LAST - G2L_REST) // L):
            g2l_v[pl.ds(G2L_LAST + G2L_REST + k * L, L)] = padv

    def compose(j, carry):
        ids = g2l_v[pl.ds(j * L, L)]
        vals = plsc.load_gather(big_v, [ids])
        vals = jnp.where(ids >= PAD, 0.0, vals)
        big_v[pl.ds(TABLE_PAD + j * L, L)] = vals
        return carry

    lax.fori_loop(0, G2L_SLICE // L, compose, 0)
    pltpu.sync_copy(big_v.at[pl.ds(TABLE_PAD, G2L_SLICE)],
                    shared.at[pl.ds(s * G2L_SLICE, G2L_SLICE)])
    plsc.subcore_barrier()
    pltpu.sync_copy(shared, big_v)
    pltpu.sync_copy(bias_hbm, bias_v)

    bv = bias_v[...]

    def pair_body(g2, carry):
        for b in (0, 1):
            g = g2 * 2 + b
            start = base_row + g * GROUP_ROWS
            pltpu.make_async_copy(
                rules_hbm.at[pl.ds(start, GROUP_ROWS)], rbufs[b],
                sems[b]).wait()
            rbuf = rbufs[b]

            def block_body(blk, carry2):
                @plsc.parallel_loop(0, L, unroll=2)
                def _(r):
                    rr = blk * L + r
                    acc = plsc.load_gather(big_v, [rbuf[rr, pl.ds(0, L)]])
                    for j in range(1, 12):
                        ids = rbuf[rr, pl.ds(j * L, L)]
                        acc = acc + plsc.load_gather(big_v, [ids])
                    tids = rbuf[rr, pl.ds(HIST - L, L)]
                    tids = jnp.where(lane >= L - (HIST - 12 * L),
                                     tids, PAD_IDX)
                    acc = acc + plsc.load_gather(big_v, [tids])
                    amat[pl.ds(r * L, L)] = acc
                tot = plsc.load_gather(amat, [lane * L])
                for j in range(1, L):
                    tot = tot + plsc.load_gather(amat, [lane * L + j])
                out_v[pl.ds(g * GROUP_ROWS + blk * L, L)] = tot + bv
                return carry2

            lax.fori_loop(0, GROUP_ROWS // L, block_body, 0)

            @pl.when(g2 < GROUPS // 2 - 1)
            def _():
                pltpu.async_copy(
                    rules_hbm.at[pl.ds(start + 2 * GROUP_ROWS, GROUP_ROWS)],
                    rbufs[b], sems[b])
        return carry

    lax.fori_loop(0, GROUPS // 2, pair_body, 0)
    pltpu.sync_copy(out_v, out_hbm.at[pl.ds(base_row, ROWS_PER_W)])


def kernel(rules, global_to_local, rules_table, bias):
    bias16 = jnp.broadcast_to(bias.reshape(()), (L,)).astype(jnp.float32)
    out = _agg(rules, global_to_local, rules_table, bias16)
    return out.reshape(BATCH, 1)


# speedup vs baseline: 1.7998x; 1.7998x over previous
import functools

import jax
import jax.numpy as jnp
from jax import lax
from jax.experimental import pallas as pl
from jax.experimental.pallas import tpu as pltpu
from jax.experimental.pallas import tpu_sc as plsc

LEN_RULES = 100000
NUM_REL_RULES = 50000
PAD = NUM_REL_RULES
BATCH = 16384
HIST = 200

NC = 2
NS = 16
NW = NC * NS
L = 16

G2L_PAD = 100352
G2L_SLICE = G2L_PAD // NS
TABLE_PAD = 50048
TABLE_MAIN = 49920
TABLE_TAIL = 80
PAD_IDX = 100001
G2L_LAST = 5888
G2L_REST = 32

ROWS_PER_W = BATCH // NW
GROUP_ROWS = 32
GROUPS = ROWS_PER_W // GROUP_ROWS
GROUP_WORDS = GROUP_ROWS * HIST

_mesh = plsc.VectorSubcoreMesh(core_axis_name="c", subcore_axis_name="s")


@functools.partial(
    pl.kernel,
    mesh=_mesh,
    compiler_params=pltpu.CompilerParams(needs_layout_passes=False),
    out_type=jax.ShapeDtypeStruct((BATCH,), jnp.float32),
    scratch_types=[
        pltpu.VMEM((G2L_PAD,), jnp.float32),
        pltpu.VMEM((GROUP_ROWS, HIST), jnp.int32),
        pltpu.VMEM((GROUP_ROWS, HIST), jnp.int32),
        pltpu.VMEM((G2L_SLICE,), jnp.int32),
        pltpu.VMEM((ROWS_PER_W,), jnp.float32),
        pltpu.VMEM((L,), jnp.float32),
        pltpu.VMEM((L * L,), jnp.float32),
        pltpu.VMEM((TABLE_TAIL,), jnp.float32),
        pltpu.VMEM((G2L_REST,), jnp.int32),
        pltpu.VMEM_SHARED((G2L_PAD,), jnp.float32),
        pltpu.SemaphoreType.DMA,
        pltpu.SemaphoreType.DMA,
    ],
)
def _agg(rules_hbm, g2l_hbm, table_hbm, bias_hbm, out_hbm,
         big_v, rbuf0, rbuf1, g2l_v, out_v, bias_v, amat, ttail, gtail,
         shared, sem0, sem1):
    c = lax.axis_index("c")
    s = lax.axis_index("s")
    wid = s * NC + c
    lane = lax.iota(jnp.int32, L)
    base_row = wid * ROWS_PER_W
    sems = (sem0, sem1)
    rbufs = (rbuf0, rbuf1)

    def rules_dma(g, b):
        start = base_row + g * GROUP_ROWS
        return pltpu.async_copy(rules_hbm.at[pl.ds(start, GROUP_ROWS)],
                                rbufs[b], sems[b])

    handles = [rules_dma(0, 0), rules_dma(1, 1)]

    pltpu.sync_copy(table_hbm.at[pl.ds(0, TABLE_MAIN)],
                    big_v.at[pl.ds(0, TABLE_MAIN)])
    pltpu.sync_copy(table_hbm.at[pl.ds(TABLE_MAIN, TABLE_TAIL)], ttail)
    for k in range(TABLE_TAIL // L):
        big_v[pl.ds(TABLE_MAIN + k * L, L)] = ttail[pl.ds(k * L, L)]

    @pl.when(s < NS - 1)
    def _():
        pltpu.sync_copy(g2l_hbm.at[pl.ds(s * G2L_SLICE, G2L_SLICE)], g2l_v)

    @pl.when(s == NS - 1)
    def _():
        base = (NS - 1) * G2L_SLICE
        pltpu.sync_copy(g2l_hbm.at[pl.ds(base, G2L_LAST)],
                        g2l_v.at[pl.ds(0, G2L_LAST)])
        pltpu.sync_copy(g2l_hbm.at[pl.ds(base + G2L_LAST, G2L_REST)], gtail)
        for k in range(G2L_REST // L):
            g2l_v[pl.ds(G2L_LAST + k * L, L)] = gtail[pl.ds(k * L, L)]
        padv = jnp.full((L,), PAD, jnp.int32)
        for k in range((G2L_SLICE - G2L_LAST - G2L_REST) // L):
            g2l_v[pl.ds(G2L_LAST + G2L_REST + k * L, L)] = padv

    def compose(j, carry):
        ids = g2l_v[pl.ds(j * L, L)]
        vals = plsc.load_gather(big_v, [ids])
        vals = jnp.where(ids >= PAD, 0.0, vals)
        big_v[pl.ds(TABLE_PAD + j * L, L)] = vals
        return carry

    lax.fori_loop(0, G2L_SLICE // L, compose, 0)
    pltpu.sync_copy(big_v.at[pl.ds(TABLE_PAD, G2L_SLICE)],
                    shared.at[pl.ds(s * G2L_SLICE, G2L_SLICE)])
    plsc.subcore_barrier()
    pltpu.sync_copy(shared, big_v)
    pltpu.sync_copy(bias_hbm, bias_v)

    bv = bias_v[...]

    def pair_body(g2, carry):
        for b in (0, 1):
            g = g2 * 2 + b
            start = base_row + g * GROUP_ROWS
            pltpu.make_async_copy(
                rules_hbm.at[pl.ds(start, GROUP_ROWS)], rbufs[b],
                sems[b]).wait()
            rbuf = rbufs[b]

            def block_body(blk, carry2):
                @plsc.parallel_loop(0, L, unroll=2)
                def _(r):
                    rr = blk * L + r
                    acc = plsc.load_gather(big_v, [rbuf[rr, pl.ds(0, L)]])
                    for j in range(1, 12):
                        ids = rbuf[rr, pl.ds(j * L, L)]
                        acc = acc + plsc.load_gather(big_v, [ids])
                    tids = rbuf[rr, pl.ds(HIST - L, L)]
                    tids = jnp.where(lane >= L - (HIST - 12 * L),
                                     tids, PAD_IDX)
                    acc = acc + plsc.load_gather(big_v, [tids])
                    amat[pl.ds(r * L, L)] = acc
                tot = plsc.load_gather(amat, [lane * L])
                for j in range(1, L):
                    tot = tot + plsc.load_gather(amat, [lane * L + j])
                out_v[pl.ds(g * GROUP_ROWS + blk * L, L)] = tot + bv
                return carry2

            lax.fori_loop(0, GROUP_ROWS // L, block_body, 0)

            @pl.when(g2 < GROUPS // 2 - 1)
            def _():
                pltpu.async_copy(
                    rules_hbm.at[pl.ds(start + 2 * GROUP_ROWS, GROUP_ROWS)],
                    rbufs[b], sems[b])
        return carry

    lax.fori_loop(0, GROUPS // 2, pair_body, 0)
    pltpu.sync_copy(out_v, out_hbm.at[pl.ds(base_row, ROWS_PER_W)])


def kernel(rules, global_to_local, rules_table, bias):
    table_flat = rules_table.reshape(-1)
    bias16 = jnp.broadcast_to(bias.reshape(()), (L,)).astype(jnp.float32)
    out = _agg(rules, global_to_local, table_flat, bias16)
    return out.reshape(BATCH, 1)
